# baseline (speedup 1.0000x reference)
import functools

import jax
import jax.numpy as jnp
from jax import lax
from jax.experimental import pallas as pl
from jax.experimental.pallas import tpu as pltpu
from jax.experimental.pallas import tpu_sc as plsc

N = 10000
H = 128
G = 128
NC = 2
NS = 16
CHUNK = 120
IDXBLK = 16
NBUF = 3
ROWS_PER_TILE = 640
NP = NS * ROWS_PER_TILE
RPTA = 632
NPA = NS * RPTA
BR = 2000
NB = N // BR

_MESH = plsc.VectorSubcoreMesh(
    core_axis_name="c", subcore_axis_name="s", num_cores=NC, num_subcores=NS)


@functools.lru_cache(maxsize=None)
def _make_deg(nch):
    def body(dst_hbm, deg_hbm, dst_v, ones_v, zeros_v, deg_shared):
        c = lax.axis_index("c")
        s = lax.axis_index("s")
        for i in range(CHUNK // 16 + 1):
            ones_v[pl.ds(i * 16, 16)] = jnp.ones((16,), jnp.float32)
        for i in range(ROWS_PER_TILE // 16):
            zeros_v[pl.ds(i * 16, 16)] = jnp.zeros((16,), jnp.float32)
        pltpu.sync_copy(zeros_v, deg_shared.at[pl.ds(s * ROWS_PER_TILE, ROWS_PER_TILE)])
        plsc.subcore_barrier()

        def outer(ib, carry):
            pltpu.sync_copy(dst_hbm.at[c, s, pl.ds(ib * IDXBLK, IDXBLK)], dst_v)
            for j in range(IDXBLK):
                pltpu.sync_copy(ones_v.at[pl.ds(0, CHUNK)],
                                deg_shared.at[dst_v.at[j]], add=True)
            return carry

        lax.fori_loop(0, nch // IDXBLK, outer, 0)
        plsc.subcore_barrier()
        pltpu.sync_copy(deg_shared.at[pl.ds(s * ROWS_PER_TILE, ROWS_PER_TILE)],
                        deg_hbm.at[c, pl.ds(s * ROWS_PER_TILE, ROWS_PER_TILE)])

    return pl.kernel(
        body,
        out_type=jax.ShapeDtypeStruct((NC, NP), jnp.float32),
        mesh=_MESH,
        scratch_types=[
            pltpu.VMEM((IDXBLK, CHUNK), jnp.int32),
            pltpu.VMEM((CHUNK + 16,), jnp.float32),
            pltpu.VMEM((ROWS_PER_TILE,), jnp.float32),
            pltpu.VMEM_SHARED((NP,), jnp.float32),
        ],
    )


@functools.lru_cache(maxsize=None)
def _make_agg(nch):
    def body(ts_hbm, src_hbm, dst_hbm, out_hbm, src_v, dst_v, rows, gsems,
             ssems, acc_shared):
        c = lax.axis_index("c")
        s = lax.axis_index("s")
        row0 = s * RPTA
        pltpu.sync_copy(ts_hbm.at[pl.ds(c * NPA + row0, RPTA)],
                        acc_shared.at[pl.ds(row0, RPTA)])
        plsc.subcore_barrier()

        def gather(j, b):
            return pltpu.async_copy(ts_hbm.at[src_v.at[j]], rows[b], gsems[b])

        def outer(ib, carry):
            pltpu.sync_copy(src_hbm.at[c, s, pl.ds(ib * IDXBLK, IDXBLK)], src_v)
            pltpu.sync_copy(dst_hbm.at[c, s, pl.ds(ib * IDXBLK, IDXBLK)], dst_v)
            scat = [None] * NBUF
            g = [None] * NBUF
            for j0 in range(NBUF - 1):
                g[j0] = gather(j0, j0)
            for j in range(IDXBLK):
                b = j % NBUF
                jn = j + NBUF - 1
                bn = jn % NBUF
                if jn < IDXBLK:
                    if scat[bn] is not None:
                        scat[bn].wait()
                    g[bn] = gather(jn, bn)
                g[b].wait()
                scat[b] = pltpu.async_copy(
                    rows[b], acc_shared.at[dst_v.at[j]], ssems[b], add=True)
            for b in range(NBUF):
                if scat[b] is not None:
                    scat[b].wait()
            return carry

        lax.fori_loop(0, nch // IDXBLK, outer, 0)
        plsc.subcore_barrier()
        pltpu.sync_copy(acc_shared.at[pl.ds(row0, RPTA)],
                        out_hbm.at[pl.ds(c * NPA + row0, RPTA)])

    return pl.kernel(
        body,
        out_type=jax.ShapeDtypeStruct((NC * NPA, H), jnp.float32),
        mesh=_MESH,
        scratch_types=[
            pltpu.VMEM((IDXBLK, CHUNK), jnp.int32),
            pltpu.VMEM((IDXBLK, CHUNK), jnp.int32),
            tuple(pltpu.VMEM((CHUNK, H), jnp.float32) for _ in range(NBUF)),
            tuple(pltpu.SemaphoreType.DMA for _ in range(NBUF)),
            tuple(pltpu.SemaphoreType.DMA for _ in range(NBUF)),
            pltpu.VMEM_SHARED((NPA, H), jnp.float32),
        ],
    )


def _prep_body(x_ref, deg_ref, win_ref, bin_ref, w1_ref, ts_ref, dinv_ref):
    dinv = lax.rsqrt(deg_ref[0] + 1.0)
    h0 = jnp.maximum(x_ref[0] * win_ref[...] + bin_ref[...], 0.0)
    t = jnp.dot(h0, w1_ref[...], preferred_element_type=jnp.float32)
    ts_ref[0] = dinv * t
    dinv_ref[0] = dinv


def _mid_body(acc_ref, dinv_ref, b_ref, w_ref, ts_ref):
    dinv = dinv_ref[0]
    h = jnp.maximum(dinv * acc_ref[0] + b_ref[...], 0.0)
    ts_ref[0] = dinv * jnp.dot(h, w_ref[...], preferred_element_type=jnp.float32)


def _pool_body(acc_ref, dinv_ref, b_ref, batch_ref, sums_ref, cnt_ref):
    i = pl.program_id(1)
    h = jnp.maximum(dinv_ref[0] * acc_ref[0] + b_ref[...], 0.0)
    onehot = (batch_ref[0] == lax.broadcasted_iota(jnp.int32, (1, G), 1)
              ).astype(jnp.float32)
    dn = (((0,), (0,)), ((), ()))
    sums_blk = lax.dot_general(onehot, h, dn, preferred_element_type=jnp.float32)
    cnt_blk = lax.dot_general(onehot, jnp.ones((BR, H), jnp.float32), dn,
                              preferred_element_type=jnp.float32)

    @pl.when(i == 0)
    def _():
        sums_ref[0] = sums_blk
        cnt_ref[0] = cnt_blk

    @pl.when(i > 0)
    def _():
        sums_ref[0] += sums_blk
        cnt_ref[0] += cnt_blk


def _head_body(sums_ref, cnt_ref, wout_ref, bout_ref, wm1_ref, bm1_ref,
               wm2_ref, bm2_ref, out_ref):
    ga = sums_ref[0] / jnp.maximum(cnt_ref[0], 1.0)
    gb = sums_ref[1] / jnp.maximum(cnt_ref[1], 1.0)
    za = jnp.maximum(jnp.dot(ga, wout_ref[...], preferred_element_type=jnp.float32)
                     + bout_ref[...], 0.0)
    zb = jnp.maximum(jnp.dot(gb, wout_ref[...], preferred_element_type=jnp.float32)
                     + bout_ref[...], 0.0)
    hid = (jnp.dot(za, wm1_ref[0:H], preferred_element_type=jnp.float32)
           + jnp.dot(zb, wm1_ref[H:2 * H], preferred_element_type=jnp.float32)
           + jnp.dot(jnp.abs(za - zb), wm1_ref[2 * H:3 * H],
                     preferred_element_type=jnp.float32)
           + jnp.dot(za * zb, wm1_ref[3 * H:4 * H],
                     preferred_element_type=jnp.float32)
           + bm1_ref[...])
    hid = jnp.maximum(hid, 0.0)
    out_ref[...] = (jnp.dot(hid, wm2_ref[...], preferred_element_type=jnp.float32)
                    + bm2_ref[...])


def _full2(shape):
    return pl.BlockSpec(shape, lambda g, i: (0, 0))


_prep_call = pl.pallas_call(
    _prep_body,
    grid=(2, NB),
    in_specs=[
        pl.BlockSpec((1, BR, 1), lambda g, i: (g, i, 0)),
        pl.BlockSpec((1, BR, 1), lambda g, i: (g, i, 0)),
        _full2((1, H)),
        _full2((1, H)),
        _full2((H, H)),
    ],
    out_specs=[
        pl.BlockSpec((1, BR, H), lambda g, i: (g, i, 0)),
        pl.BlockSpec((1, BR, 1), lambda g, i: (g, i, 0)),
    ],
    out_shape=[
        jax.ShapeDtypeStruct((2, NPA, H), jnp.float32),
        jax.ShapeDtypeStruct((2, N, 1), jnp.float32),
    ],
    compiler_params=pltpu.CompilerParams(
        dimension_semantics=("parallel", "parallel")),
)

_mid_call = pl.pallas_call(
    _mid_body,
    grid=(2, NB),
    in_specs=[
        pl.BlockSpec((1, BR, H), lambda g, i: (g, i, 0)),
        pl.BlockSpec((1, BR, 1), lambda g, i: (g, i, 0)),
        _full2((1, H)),
        _full2((H, H)),
    ],
    out_specs=pl.BlockSpec((1, BR, H), lambda g, i: (g, i, 0)),
    out_shape=jax.ShapeDtypeStruct((2, NPA, H), jnp.float32),
    compiler_params=pltpu.CompilerParams(
        dimension_semantics=("parallel", "parallel")),
)

_pool_call = pl.pallas_call(
    _pool_body,
    grid=(2, NB),
    in_specs=[
        pl.BlockSpec((1, BR, H), lambda g, i: (g, i, 0)),
        pl.BlockSpec((1, BR, 1), lambda g, i: (g, i, 0)),
        _full2((1, H)),
        pl.BlockSpec((1, BR, 1), lambda g, i: (g, i, 0)),
    ],
    out_specs=[
        pl.BlockSpec((1, G, H), lambda g, i: (g, 0, 0)),
        pl.BlockSpec((1, G, H), lambda g, i: (g, 0, 0)),
    ],
    out_shape=[
        jax.ShapeDtypeStruct((2, G, H), jnp.float32),
        jax.ShapeDtypeStruct((2, G, H), jnp.float32),
    ],
    compiler_params=pltpu.CompilerParams(
        dimension_semantics=("parallel", "arbitrary")),
)

_head_call = pl.pallas_call(
    _head_body,
    out_shape=jax.ShapeDtypeStruct((G, 1), jnp.float32),
)


def _prep_edges(edge_index, core, e_pad, nch):
    e = edge_index.shape[1]
    pad = e_pad - e
    src = jnp.concatenate(
        [edge_index[0], jnp.zeros((pad,), jnp.int32)]) + core * NPA
    dst = jnp.concatenate([edge_index[1], jnp.full((pad,), N, jnp.int32)])
    return src.reshape(NS, nch, CHUNK), dst.reshape(NS, nch, CHUNK)


def kernel(x_a, edge_index_a, batch_a, x_b, edge_index_b, batch_b,
           W_in, b_in, W1, b1, W2, b2, W3, b3, W_out, b_out,
           W_m1, b_m1, W_m2, b_m2):
    e = edge_index_a.shape[1]
    nch = -(-e // (NS * CHUNK))
    nch = -(-nch // IDXBLK) * IDXBLK
    e_pad = NS * nch * CHUNK

    src_a, dst_a = _prep_edges(edge_index_a, 0, e_pad, nch)
    src_b, dst_b = _prep_edges(edge_index_b, 1, e_pad, nch)
    src_all = jnp.stack([src_a, src_b])
    dst_all = jnp.stack([dst_a, dst_b])

    deg = _make_deg(nch)(dst_all)
    deg_n = deg.reshape(2, NP, 1)[:, :N, :]
    x_s = jnp.stack([x_a, x_b]).reshape(2, N, 1)

    agg = _make_agg(nch)
    ts1, dinv = _prep_call(x_s, deg_n, W_in.reshape(1, H),
                           b_in.reshape(1, H), W1)
    acc1 = agg(ts1.reshape(2 * NPA, H), src_all, dst_all)
    ts2 = _mid_call(acc1.reshape(2, NPA, H), dinv, b1.reshape(1, H), W2)
    acc2 = agg(ts2.reshape(2 * NPA, H), src_all, dst_all)
    ts3 = _mid_call(acc2.reshape(2, NPA, H), dinv, b2.reshape(1, H), W3)
    acc3 = agg(ts3.reshape(2 * NPA, H), src_all, dst_all)

    batch_s = jnp.stack([batch_a, batch_b]).reshape(2, N, 1)
    sums, cnt = _pool_call(acc3.reshape(2, NPA, H), dinv,
                           b3.reshape(1, H), batch_s)
    out = _head_call(sums, cnt, W_out, b_out.reshape(1, H),
                     W_m1, b_m1.reshape(1, H), W_m2, b_m2.reshape(1, 1))
    return out.reshape(G)


# speedup vs baseline: 2.6513x; 2.6513x over previous
import functools

import jax
import jax.numpy as jnp
from jax import lax
from jax.experimental import pallas as pl
from jax.experimental.pallas import tpu as pltpu
from jax.experimental.pallas import tpu_sc as plsc

N = 10000
H = 128
G = 128
NC = 2
NS = 16
CHUNK = 120
IDXBLK = 8
NBUF = 3
ROWS_PER_TILE = 640
NP = NS * ROWS_PER_TILE
RPTA = 632
NPA = NS * RPTA
BR = 2000
NB = N // BR

_MESH = plsc.VectorSubcoreMesh(
    core_axis_name="c", subcore_axis_name="s", num_cores=NC, num_subcores=NS)


@functools.lru_cache(maxsize=None)
def _make_deg(nch):
    def body(dst_hbm, deg_hbm, dst_v, ones_v, zeros_v, deg_shared):
        c = lax.axis_index("c")
        s = lax.axis_index("s")
        for i in range(CHUNK // 16 + 1):
            ones_v[pl.ds(i * 16, 16)] = jnp.ones((16,), jnp.float32)
        for i in range(ROWS_PER_TILE // 16):
            zeros_v[pl.ds(i * 16, 16)] = jnp.zeros((16,), jnp.float32)
        pltpu.sync_copy(zeros_v, deg_shared.at[pl.ds(s * ROWS_PER_TILE, ROWS_PER_TILE)])
        plsc.subcore_barrier()

        def outer(ib, carry):
            pltpu.sync_copy(dst_hbm.at[c, s, pl.ds(ib * IDXBLK, IDXBLK)], dst_v)
            for j in range(IDXBLK):
                pltpu.sync_copy(ones_v.at[pl.ds(0, CHUNK)],
                                deg_shared.at[dst_v.at[j]], add=True)
            return carry

        lax.fori_loop(0, nch // IDXBLK, outer, 0)
        plsc.subcore_barrier()
        pltpu.sync_copy(deg_shared.at[pl.ds(s * ROWS_PER_TILE, ROWS_PER_TILE)],
                        deg_hbm.at[c, pl.ds(s * ROWS_PER_TILE, ROWS_PER_TILE)])

    return pl.kernel(
        body,
        out_type=jax.ShapeDtypeStruct((NC, NP), jnp.float32),
        mesh=_MESH,
        scratch_types=[
            pltpu.VMEM((IDXBLK, CHUNK), jnp.int32),
            pltpu.VMEM((CHUNK + 16,), jnp.float32),
            pltpu.VMEM((ROWS_PER_TILE,), jnp.float32),
            pltpu.VMEM_SHARED((NP,), jnp.float32),
        ],
    )


@functools.lru_cache(maxsize=None)
def _make_agg(nch):
    def body(ts_hbm, src_hbm, dst_hbm, out_hbm, src_v, dst_v, rows, gsems,
             ssems, acc_shared):
        c = lax.axis_index("c")
        s = lax.axis_index("s")
        row0 = s * RPTA
        pltpu.sync_copy(ts_hbm.at[pl.ds(c * NPA + row0, RPTA)],
                        acc_shared.at[pl.ds(row0, RPTA)])
        plsc.subcore_barrier()

        def gather(j, b):
            return pltpu.async_copy(ts_hbm.at[src_v.at[j]], rows[b], gsems[b])

        def outer(ib, carry):
            pltpu.sync_copy(src_hbm.at[c, s, pl.ds(ib * IDXBLK, IDXBLK)], src_v)
            pltpu.sync_copy(dst_hbm.at[c, s, pl.ds(ib * IDXBLK, IDXBLK)], dst_v)
            scat = [None] * NBUF
            g = [None] * NBUF
            for j0 in range(NBUF - 1):
                g[j0] = gather(j0, j0)
            for j in range(IDXBLK):
                b = j % NBUF
                jn = j + NBUF - 1
                bn = jn % NBUF
                if jn < IDXBLK:
                    if scat[bn] is not None:
                        scat[bn].wait()
                    g[bn] = gather(jn, bn)
                g[b].wait()
                scat[b] = pltpu.async_copy(
                    rows[b], acc_shared.at[dst_v.at[j]], ssems[b], add=True)
            for b in range(NBUF):
                if scat[b] is not None:
                    scat[b].wait()
            return carry

        lax.fori_loop(0, nch // IDXBLK, outer, 0)
        plsc.subcore_barrier()
        pltpu.sync_copy(acc_shared.at[pl.ds(row0, RPTA)],
                        out_hbm.at[pl.ds(c * NPA + row0, RPTA)])

    return pl.kernel(
        body,
        out_type=jax.ShapeDtypeStruct((NC * NPA, H), jnp.float32),
        mesh=_MESH,
        scratch_types=[
            pltpu.VMEM((IDXBLK, CHUNK), jnp.int32),
            pltpu.VMEM((IDXBLK, CHUNK), jnp.int32),
            tuple(pltpu.VMEM((CHUNK, H), jnp.float32) for _ in range(NBUF)),
            tuple(pltpu.SemaphoreType.DMA for _ in range(NBUF)),
            tuple(pltpu.SemaphoreType.DMA for _ in range(NBUF)),
            pltpu.VMEM_SHARED((NPA, H), jnp.float32),
        ],
    )


def _prep_body(x_ref, deg_ref, win_ref, bin_ref, w1_ref, ts_ref, dinv_ref):
    dinv = lax.rsqrt(deg_ref[0] + 1.0)
    h0 = jnp.maximum(x_ref[0] * win_ref[...] + bin_ref[...], 0.0)
    t = jnp.dot(h0, w1_ref[...], preferred_element_type=jnp.float32)
    ts_ref[0] = dinv * t
    dinv_ref[0] = dinv


def _mid_body(acc_ref, dinv_ref, b_ref, w_ref, ts_ref):
    dinv = dinv_ref[0]
    h = jnp.maximum(dinv * acc_ref[0] + b_ref[...], 0.0)
    ts_ref[0] = dinv * jnp.dot(h, w_ref[...], preferred_element_type=jnp.float32)


def _pool_body(acc_ref, dinv_ref, b_ref, batch_ref, sums_ref, cnt_ref):
    i = pl.program_id(1)
    h = jnp.maximum(dinv_ref[0] * acc_ref[0] + b_ref[...], 0.0)
    onehot = (batch_ref[0] == lax.broadcasted_iota(jnp.int32, (1, G), 1)
              ).astype(jnp.float32)
    dn = (((0,), (0,)), ((), ()))
    sums_blk = lax.dot_general(onehot, h, dn, preferred_element_type=jnp.float32)
    cnt_blk = lax.dot_general(onehot, jnp.ones((BR, H), jnp.float32), dn,
                              preferred_element_type=jnp.float32)

    @pl.when(i == 0)
    def _():
        sums_ref[0] = sums_blk
        cnt_ref[0] = cnt_blk

    @pl.when(i > 0)
    def _():
        sums_ref[0] += sums_blk
        cnt_ref[0] += cnt_blk


def _head_body(sums_ref, cnt_ref, wout_ref, bout_ref, wm1_ref, bm1_ref,
               wm2_ref, bm2_ref, out_ref):
    ga = sums_ref[0] / jnp.maximum(cnt_ref[0], 1.0)
    gb = sums_ref[1] / jnp.maximum(cnt_ref[1], 1.0)
    za = jnp.maximum(jnp.dot(ga, wout_ref[...], preferred_element_type=jnp.float32)
                     + bout_ref[...], 0.0)
    zb = jnp.maximum(jnp.dot(gb, wout_ref[...], preferred_element_type=jnp.float32)
                     + bout_ref[...], 0.0)
    hid = (jnp.dot(za, wm1_ref[0:H], preferred_element_type=jnp.float32)
           + jnp.dot(zb, wm1_ref[H:2 * H], preferred_element_type=jnp.float32)
           + jnp.dot(jnp.abs(za - zb), wm1_ref[2 * H:3 * H],
                     preferred_element_type=jnp.float32)
           + jnp.dot(za * zb, wm1_ref[3 * H:4 * H],
                     preferred_element_type=jnp.float32)
           + bm1_ref[...])
    hid = jnp.maximum(hid, 0.0)
    out_ref[...] = (jnp.dot(hid, wm2_ref[...], preferred_element_type=jnp.float32)
                    + bm2_ref[...])


def _full2(shape):
    return pl.BlockSpec(shape, lambda g, i: (0, 0))


_prep_call = pl.pallas_call(
    _prep_body,
    grid=(2, NB),
    in_specs=[
        pl.BlockSpec((1, BR, 1), lambda g, i: (g, i, 0)),
        pl.BlockSpec((1, BR, 1), lambda g, i: (g, i, 0)),
        _full2((1, H)),
        _full2((1, H)),
        _full2((H, H)),
    ],
    out_specs=[
        pl.BlockSpec((1, BR, H), lambda g, i: (g, i, 0)),
        pl.BlockSpec((1, BR, 1), lambda g, i: (g, i, 0)),
    ],
    out_shape=[
        jax.ShapeDtypeStruct((2, NPA, H), jnp.float32),
        jax.ShapeDtypeStruct((2, N, 1), jnp.float32),
    ],
    compiler_params=pltpu.CompilerParams(
        dimension_semantics=("parallel", "parallel")),
)

_mid_call = pl.pallas_call(
    _mid_body,
    grid=(2, NB),
    in_specs=[
        pl.BlockSpec((1, BR, H), lambda g, i: (g, i, 0)),
        pl.BlockSpec((1, BR, 1), lambda g, i: (g, i, 0)),
        _full2((1, H)),
        _full2((H, H)),
    ],
    out_specs=pl.BlockSpec((1, BR, H), lambda g, i: (g, i, 0)),
    out_shape=jax.ShapeDtypeStruct((2, NPA, H), jnp.float32),
    compiler_params=pltpu.CompilerParams(
        dimension_semantics=("parallel", "parallel")),
)

_pool_call = pl.pallas_call(
    _pool_body,
    grid=(2, NB),
    in_specs=[
        pl.BlockSpec((1, BR, H), lambda g, i: (g, i, 0)),
        pl.BlockSpec((1, BR, 1), lambda g, i: (g, i, 0)),
        _full2((1, H)),
        pl.BlockSpec((1, BR, 1), lambda g, i: (g, i, 0)),
    ],
    out_specs=[
        pl.BlockSpec((1, G, H), lambda g, i: (g, 0, 0)),
        pl.BlockSpec((1, G, H), lambda g, i: (g, 0, 0)),
    ],
    out_shape=[
        jax.ShapeDtypeStruct((2, G, H), jnp.float32),
        jax.ShapeDtypeStruct((2, G, H), jnp.float32),
    ],
    compiler_params=pltpu.CompilerParams(
        dimension_semantics=("parallel", "arbitrary")),
)

_head_call = pl.pallas_call(
    _head_body,
    out_shape=jax.ShapeDtypeStruct((G, 1), jnp.float32),
)


def _prep_edges(edge_index, core, e_pad, nch):
    e = edge_index.shape[1]
    pad = e_pad - e
    src = jnp.concatenate(
        [edge_index[0], jnp.zeros((pad,), jnp.int32)]) + core * NPA
    dst = jnp.concatenate([edge_index[1], jnp.full((pad,), N, jnp.int32)])
    return src.reshape(NS, nch, CHUNK), dst.reshape(NS, nch, CHUNK)


def kernel(x_a, edge_index_a, batch_a, x_b, edge_index_b, batch_b,
           W_in, b_in, W1, b1, W2, b2, W3, b3, W_out, b_out,
           W_m1, b_m1, W_m2, b_m2):
    e = edge_index_a.shape[1]
    nch = -(-e // (NS * CHUNK))
    nch = -(-nch // IDXBLK) * IDXBLK
    e_pad = NS * nch * CHUNK

    src_a, dst_a = _prep_edges(edge_index_a, 0, e_pad, nch)
    src_b, dst_b = _prep_edges(edge_index_b, 1, e_pad, nch)
    src_all = jnp.stack([src_a, src_b])
    dst_all = jnp.stack([dst_a, dst_b])

    deg = _make_deg(nch)(dst_all)
    deg_n = deg.reshape(2, NP, 1)[:, :N, :]
    x_s = jnp.stack([x_a, x_b]).reshape(2, N, 1)

    agg = _make_agg(nch)
    ts1, dinv = _prep_call(x_s, deg_n, W_in.reshape(1, H),
                           b_in.reshape(1, H), W1)
    acc1 = agg(ts1.reshape(2 * NPA, H), src_all, dst_all)
    ts2 = _mid_call(acc1.reshape(2, NPA, H), dinv, b1.reshape(1, H), W2)
    acc2 = agg(ts2.reshape(2 * NPA, H), src_all, dst_all)
    ts3 = _mid_call(acc2.reshape(2, NPA, H), dinv, b2.reshape(1, H), W3)
    acc3 = agg(ts3.reshape(2 * NPA, H), src_all, dst_all)

    batch_s = jnp.stack([batch_a, batch_b]).reshape(2, N, 1)
    sums, cnt = _pool_call(acc3.reshape(2, NPA, H), dinv,
                           b3.reshape(1, H), batch_s)
    out = _head_call(sums, cnt, W_out, b_out.reshape(1, H),
                     W_m1, b_m1.reshape(1, H), W_m2, b_m2.reshape(1, 1))
    return out.reshape(G)


# speedup vs baseline: 2.7123x; 1.0230x over previous
import functools

import jax
import jax.numpy as jnp
from jax import lax
from jax.experimental import pallas as pl
from jax.experimental.pallas import tpu as pltpu
from jax.experimental.pallas import tpu_sc as plsc

N = 10000
H = 128
G = 128
NC = 2
NS = 16
CHUNK = 120
IDXBLK = 8
NBUF = 3
ROWS_PER_TILE = 640
NP = NS * ROWS_PER_TILE
RPTA = 632
NPA = NS * RPTA
BR = 2000
NB = N // BR

_MESH = plsc.VectorSubcoreMesh(
    core_axis_name="c", subcore_axis_name="s", num_cores=NC, num_subcores=NS)


@functools.lru_cache(maxsize=None)
def _make_deg(nch):
    def body(dst_hbm, deg_hbm, dst_v, ones_v, zeros_v, deg_shared):
        c = lax.axis_index("c")
        s = lax.axis_index("s")
        for i in range(CHUNK // 16 + 1):
            ones_v[pl.ds(i * 16, 16)] = jnp.ones((16,), jnp.float32)
        for i in range(ROWS_PER_TILE // 16):
            zeros_v[pl.ds(i * 16, 16)] = jnp.zeros((16,), jnp.float32)
        pltpu.sync_copy(zeros_v, deg_shared.at[pl.ds(s * ROWS_PER_TILE, ROWS_PER_TILE)])
        plsc.subcore_barrier()

        def outer(ib, carry):
            pltpu.sync_copy(dst_hbm.at[c, s, pl.ds(ib * IDXBLK, IDXBLK)], dst_v)
            for j in range(IDXBLK):
                pltpu.sync_copy(ones_v.at[pl.ds(0, CHUNK)],
                                deg_shared.at[dst_v.at[j]], add=True)
            return carry

        lax.fori_loop(0, nch // IDXBLK, outer, 0)
        plsc.subcore_barrier()
        pltpu.sync_copy(deg_shared.at[pl.ds(s * ROWS_PER_TILE, ROWS_PER_TILE)],
                        deg_hbm.at[c, pl.ds(s * ROWS_PER_TILE, ROWS_PER_TILE)])

    return pl.kernel(
        body,
        out_type=jax.ShapeDtypeStruct((NC, NP), jnp.float32),
        mesh=_MESH,
        scratch_types=[
            pltpu.VMEM((IDXBLK, CHUNK), jnp.int32),
            pltpu.VMEM((CHUNK + 16,), jnp.float32),
            pltpu.VMEM((ROWS_PER_TILE,), jnp.float32),
            pltpu.VMEM_SHARED((NP,), jnp.float32),
        ],
    )


@functools.lru_cache(maxsize=None)
def _make_agg(nch):
    def body(ts_hbm, src_hbm, dst_hbm, out_hbm, src_v, dst_v, rows, gsems,
             ssems, isem0, isem1, acc_shared):
        c = lax.axis_index("c")
        s = lax.axis_index("s")
        row0 = s * RPTA
        pltpu.sync_copy(ts_hbm.at[pl.ds(c * NPA + row0, RPTA)],
                        acc_shared.at[pl.ds(row0, RPTA)])
        plsc.subcore_barrier()

        def gather(j, b):
            return pltpu.async_copy(ts_hbm.at[src_v.at[j]], rows[b], gsems[b])

        def outer(ib, carry):
            i1 = pltpu.async_copy(
                src_hbm.at[c, s, pl.ds(ib * IDXBLK, IDXBLK)], src_v, isem0)
            i2 = pltpu.async_copy(
                dst_hbm.at[c, s, pl.ds(ib * IDXBLK, IDXBLK)], dst_v, isem1)
            i1.wait()
            i2.wait()
            scat = [None] * NBUF
            g = [None] * NBUF
            for j0 in range(NBUF - 1):
                g[j0] = gather(j0, j0)
            for j in range(IDXBLK):
                b = j % NBUF
                jn = j + NBUF - 1
                bn = jn % NBUF
                if jn < IDXBLK:
                    if scat[bn] is not None:
                        scat[bn].wait()
                    g[bn] = gather(jn, bn)
                g[b].wait()
                scat[b] = pltpu.async_copy(
                    rows[b], acc_shared.at[dst_v.at[j]], ssems[b], add=True)
            for b in range(NBUF):
                if scat[b] is not None:
                    scat[b].wait()
            return carry

        lax.fori_loop(0, nch // IDXBLK, outer, 0)
        plsc.subcore_barrier()
        pltpu.sync_copy(acc_shared.at[pl.ds(row0, RPTA)],
                        out_hbm.at[pl.ds(c * NPA + row0, RPTA)])

    return pl.kernel(
        body,
        out_type=jax.ShapeDtypeStruct((NC * NPA, H), jnp.float32),
        mesh=_MESH,
        scratch_types=[
            pltpu.VMEM((IDXBLK, CHUNK), jnp.int32),
            pltpu.VMEM((IDXBLK, CHUNK), jnp.int32),
            tuple(pltpu.VMEM((CHUNK, H), jnp.float32) for _ in range(NBUF)),
            tuple(pltpu.SemaphoreType.DMA for _ in range(NBUF)),
            tuple(pltpu.SemaphoreType.DMA for _ in range(NBUF)),
            pltpu.SemaphoreType.DMA,
            pltpu.SemaphoreType.DMA,
            pltpu.VMEM_SHARED((NPA, H), jnp.float32),
        ],
    )


def _prep_body(x_ref, deg_ref, win_ref, bin_ref, w1_ref, ts_ref, dinv_ref):
    dinv = lax.rsqrt(deg_ref[0] + 1.0)
    h0 = jnp.maximum(x_ref[0] * win_ref[...] + bin_ref[...], 0.0)
    t = jnp.dot(h0, w1_ref[...], preferred_element_type=jnp.float32)
    ts_ref[0] = dinv * t
    dinv_ref[0] = dinv


def _mid_body(acc_ref, dinv_ref, b_ref, w_ref, ts_ref):
    dinv = dinv_ref[0]
    h = jnp.maximum(dinv * acc_ref[0] + b_ref[...], 0.0)
    ts_ref[0] = dinv * jnp.dot(h, w_ref[...], preferred_element_type=jnp.float32)


def _pool_body(acc_ref, dinv_ref, b_ref, batch_ref, sums_ref, cnt_ref):
    i = pl.program_id(1)
    h = jnp.maximum(dinv_ref[0] * acc_ref[0] + b_ref[...], 0.0)
    onehot = (batch_ref[0] == lax.broadcasted_iota(jnp.int32, (1, G), 1)
              ).astype(jnp.float32)
    dn = (((0,), (0,)), ((), ()))
    sums_blk = lax.dot_general(onehot, h, dn, preferred_element_type=jnp.float32)
    cnt_blk = lax.dot_general(onehot, jnp.ones((BR, H), jnp.float32), dn,
                              preferred_element_type=jnp.float32)

    @pl.when(i == 0)
    def _():
        sums_ref[0] = sums_blk
        cnt_ref[0] = cnt_blk

    @pl.when(i > 0)
    def _():
        sums_ref[0] += sums_blk
        cnt_ref[0] += cnt_blk


def _head_body(sums_ref, cnt_ref, wout_ref, bout_ref, wm1_ref, bm1_ref,
               wm2_ref, bm2_ref, out_ref):
    ga = sums_ref[0] / jnp.maximum(cnt_ref[0], 1.0)
    gb = sums_ref[1] / jnp.maximum(cnt_ref[1], 1.0)
    za = jnp.maximum(jnp.dot(ga, wout_ref[...], preferred_element_type=jnp.float32)
                     + bout_ref[...], 0.0)
    zb = jnp.maximum(jnp.dot(gb, wout_ref[...], preferred_element_type=jnp.float32)
                     + bout_ref[...], 0.0)
    hid = (jnp.dot(za, wm1_ref[0:H], preferred_element_type=jnp.float32)
           + jnp.dot(zb, wm1_ref[H:2 * H], preferred_element_type=jnp.float32)
           + jnp.dot(jnp.abs(za - zb), wm1_ref[2 * H:3 * H],
                     preferred_element_type=jnp.float32)
           + jnp.dot(za * zb, wm1_ref[3 * H:4 * H],
                     preferred_element_type=jnp.float32)
           + bm1_ref[...])
    hid = jnp.maximum(hid, 0.0)
    out_ref[...] = (jnp.dot(hid, wm2_ref[...], preferred_element_type=jnp.float32)
                    + bm2_ref[...])


def _full2(shape):
    return pl.BlockSpec(shape, lambda g, i: (0, 0))


_prep_call = pl.pallas_call(
    _prep_body,
    grid=(2, NB),
    in_specs=[
        pl.BlockSpec((1, BR, 1), lambda g, i: (g, i, 0)),
        pl.BlockSpec((1, BR, 1), lambda g, i: (g, i, 0)),
        _full2((1, H)),
        _full2((1, H)),
        _full2((H, H)),
    ],
    out_specs=[
        pl.BlockSpec((1, BR, H), lambda g, i: (g, i, 0)),
        pl.BlockSpec((1, BR, 1), lambda g, i: (g, i, 0)),
    ],
    out_shape=[
        jax.ShapeDtypeStruct((2, NPA, H), jnp.float32),
        jax.ShapeDtypeStruct((2, N, 1), jnp.float32),
    ],
    compiler_params=pltpu.CompilerParams(
        dimension_semantics=("parallel", "parallel")),
)

_mid_call = pl.pallas_call(
    _mid_body,
    grid=(2, NB),
    in_specs=[
        pl.BlockSpec((1, BR, H), lambda g, i: (g, i, 0)),
        pl.BlockSpec((1, BR, 1), lambda g, i: (g, i, 0)),
        _full2((1, H)),
        _full2((H, H)),
    ],
    out_specs=pl.BlockSpec((1, BR, H), lambda g, i: (g, i, 0)),
    out_shape=jax.ShapeDtypeStruct((2, NPA, H), jnp.float32),
    compiler_params=pltpu.CompilerParams(
        dimension_semantics=("parallel", "parallel")),
)

_pool_call = pl.pallas_call(
    _pool_body,
    grid=(2, NB),
    in_specs=[
        pl.BlockSpec((1, BR, H), lambda g, i: (g, i, 0)),
        pl.BlockSpec((1, BR, 1), lambda g, i: (g, i, 0)),
        _full2((1, H)),
        pl.BlockSpec((1, BR, 1), lambda g, i: (g, i, 0)),
    ],
    out_specs=[
        pl.BlockSpec((1, G, H), lambda g, i: (g, 0, 0)),
        pl.BlockSpec((1, G, H), lambda g, i: (g, 0, 0)),
    ],
    out_shape=[
        jax.ShapeDtypeStruct((2, G, H), jnp.float32),
        jax.ShapeDtypeStruct((2, G, H), jnp.float32),
    ],
    compiler_params=pltpu.CompilerParams(
        dimension_semantics=("parallel", "arbitrary")),
)

_head_call = pl.pallas_call(
    _head_body,
    out_shape=jax.ShapeDtypeStruct((G, 1), jnp.float32),
)


def _prep_edges(edge_index, core, e_pad, nch):
    e = edge_index.shape[1]
    pad = e_pad - e
    src = jnp.concatenate(
        [edge_index[0], jnp.zeros((pad,), jnp.int32)]) + core * NPA
    dst = jnp.concatenate([edge_index[1], jnp.full((pad,), N, jnp.int32)])
    return src.reshape(NS, nch, CHUNK), dst.reshape(NS, nch, CHUNK)


def kernel(x_a, edge_index_a, batch_a, x_b, edge_index_b, batch_b,
           W_in, b_in, W1, b1, W2, b2, W3, b3, W_out, b_out,
           W_m1, b_m1, W_m2, b_m2):
    e = edge_index_a.shape[1]
    nch = -(-e // (NS * CHUNK))
    nch = -(-nch // IDXBLK) * IDXBLK
    e_pad = NS * nch * CHUNK

    src_a, dst_a = _prep_edges(edge_index_a, 0, e_pad, nch)
    src_b, dst_b = _prep_edges(edge_index_b, 1, e_pad, nch)
    src_all = jnp.stack([src_a, src_b])
    dst_all = jnp.stack([dst_a, dst_b])

    deg = _make_deg(nch)(dst_all)
    deg_n = deg.reshape(2, NP, 1)[:, :N, :]
    x_s = jnp.stack([x_a, x_b]).reshape(2, N, 1)

    agg = _make_agg(nch)
    ts1, dinv = _prep_call(x_s, deg_n, W_in.reshape(1, H),
                           b_in.reshape(1, H), W1)
    acc1 = agg(ts1.reshape(2 * NPA, H), src_all, dst_all)
    ts2 = _mid_call(acc1.reshape(2, NPA, H), dinv, b1.reshape(1, H), W2)
    acc2 = agg(ts2.reshape(2 * NPA, H), src_all, dst_all)
    ts3 = _mid_call(acc2.reshape(2, NPA, H), dinv, b2.reshape(1, H), W3)
    acc3 = agg(ts3.reshape(2 * NPA, H), src_all, dst_all)

    batch_s = jnp.stack([batch_a, batch_b]).reshape(2, N, 1)
    sums, cnt = _pool_call(acc3.reshape(2, NPA, H), dinv,
                           b3.reshape(1, H), batch_s)
    out = _head_call(sums, cnt, W_out, b_out.reshape(1, H),
                     W_m1, b_m1.reshape(1, H), W_m2, b_m2.reshape(1, 1))
    return out.reshape(G)


# speedup vs baseline: 2.7816x; 1.0255x over previous
import functools

import jax
import jax.numpy as jnp
from jax import lax
from jax.experimental import pallas as pl
from jax.experimental.pallas import tpu as pltpu
from jax.experimental.pallas import tpu_sc as plsc

N = 10000
H = 128
G = 128
NC = 2
NS = 16
CHUNK = 120
IDXBLK = 8
NBUF = 3
ROWS_PER_TILE = 640
NP = NS * ROWS_PER_TILE
RPTA = 632
NPA = NS * RPTA
BR = 2000
NB = N // BR

_MESH = plsc.VectorSubcoreMesh(
    core_axis_name="c", subcore_axis_name="s", num_cores=NC, num_subcores=NS)


@functools.lru_cache(maxsize=None)
def _make_deg(nch):
    def body(dst_hbm, deg_hbm, dst_v, ones_v, zeros_v, deg_shared):
        c = lax.axis_index("c")
        s = lax.axis_index("s")
        for i in range(CHUNK // 16 + 1):
            ones_v[pl.ds(i * 16, 16)] = jnp.ones((16,), jnp.float32)
        for i in range(ROWS_PER_TILE // 16):
            zeros_v[pl.ds(i * 16, 16)] = jnp.zeros((16,), jnp.float32)
        pltpu.sync_copy(zeros_v, deg_shared.at[pl.ds(s * ROWS_PER_TILE, ROWS_PER_TILE)])
        plsc.subcore_barrier()

        def outer(ib, carry):
            pltpu.sync_copy(dst_hbm.at[c, s, pl.ds(ib * IDXBLK, IDXBLK)], dst_v)
            for j in range(IDXBLK):
                pltpu.sync_copy(ones_v.at[pl.ds(0, CHUNK)],
                                deg_shared.at[dst_v.at[j]], add=True)
            return carry

        lax.fori_loop(0, nch // IDXBLK, outer, 0)
        plsc.subcore_barrier()
        pltpu.sync_copy(deg_shared.at[pl.ds(s * ROWS_PER_TILE, ROWS_PER_TILE)],
                        deg_hbm.at[c, pl.ds(s * ROWS_PER_TILE, ROWS_PER_TILE)])

    return pl.kernel(
        body,
        out_type=jax.ShapeDtypeStruct((NC, NP), jnp.float32),
        mesh=_MESH,
        scratch_types=[
            pltpu.VMEM((IDXBLK, CHUNK), jnp.int32),
            pltpu.VMEM((CHUNK + 16,), jnp.float32),
            pltpu.VMEM((ROWS_PER_TILE,), jnp.float32),
            pltpu.VMEM_SHARED((NP,), jnp.float32),
        ],
    )


@functools.lru_cache(maxsize=None)
def _make_agg(nch):
    def body(ts_hbm, src_hbm, dst_hbm, out_hbm, src_v, dst_v, rows, gsems,
             ssems, isem0, isem1, acc_shared):
        c = lax.axis_index("c")
        s = lax.axis_index("s")
        row0 = s * RPTA
        pltpu.sync_copy(ts_hbm.at[pl.ds(c * NPA + row0, RPTA)],
                        acc_shared.at[pl.ds(row0, RPTA)])
        plsc.subcore_barrier()

        def gather(j, b):
            return pltpu.async_copy(ts_hbm.at[src_v.at[j]], rows[b], gsems[b])

        def drain(slot):
            pltpu.make_async_copy(
                rows[slot], acc_shared.at[dst_v.at[0]], ssems[slot]).wait()

        def outer(ib, carry):
            i1 = pltpu.async_copy(
                src_hbm.at[c, s, pl.ds(ib * IDXBLK, IDXBLK)], src_v, isem0)
            i2 = pltpu.async_copy(
                dst_hbm.at[c, s, pl.ds(ib * IDXBLK, IDXBLK)], dst_v, isem1)

            @pl.when(ib > 0)
            def _():
                for slot in range(NBUF):
                    drain(slot)

            i1.wait()
            i2.wait()
            scat = [None] * NBUF
            g = [None] * NBUF
            for j0 in range(NBUF - 1):
                g[j0] = gather(j0, j0)
            for j in range(IDXBLK):
                b = j % NBUF
                jn = j + NBUF - 1
                bn = jn % NBUF
                if jn < IDXBLK:
                    if scat[bn] is not None:
                        scat[bn].wait()
                    g[bn] = gather(jn, bn)
                g[b].wait()
                scat[b] = pltpu.async_copy(
                    rows[b], acc_shared.at[dst_v.at[j]], ssems[b], add=True)
            return carry

        lax.fori_loop(0, nch // IDXBLK, outer, 0)
        for slot in range(NBUF):
            drain(slot)
        plsc.subcore_barrier()
        pltpu.sync_copy(acc_shared.at[pl.ds(row0, RPTA)],
                        out_hbm.at[pl.ds(c * NPA + row0, RPTA)])

    return pl.kernel(
        body,
        out_type=jax.ShapeDtypeStruct((NC * NPA, H), jnp.float32),
        mesh=_MESH,
        scratch_types=[
            pltpu.VMEM((IDXBLK, CHUNK), jnp.int32),
            pltpu.VMEM((IDXBLK, CHUNK), jnp.int32),
            tuple(pltpu.VMEM((CHUNK, H), jnp.float32) for _ in range(NBUF)),
            tuple(pltpu.SemaphoreType.DMA for _ in range(NBUF)),
            tuple(pltpu.SemaphoreType.DMA for _ in range(NBUF)),
            pltpu.SemaphoreType.DMA,
            pltpu.SemaphoreType.DMA,
            pltpu.VMEM_SHARED((NPA, H), jnp.float32),
        ],
    )


def _prep_body(x_ref, deg_ref, win_ref, bin_ref, w1_ref, ts_ref, dinv_ref):
    dinv = lax.rsqrt(deg_ref[0] + 1.0)
    h0 = jnp.maximum(x_ref[0] * win_ref[...] + bin_ref[...], 0.0)
    t = jnp.dot(h0, w1_ref[...], preferred_element_type=jnp.float32)
    ts_ref[0] = dinv * t
    dinv_ref[0] = dinv


def _mid_body(acc_ref, dinv_ref, b_ref, w_ref, ts_ref):
    dinv = dinv_ref[0]
    h = jnp.maximum(dinv * acc_ref[0] + b_ref[...], 0.0)
    ts_ref[0] = dinv * jnp.dot(h, w_ref[...], preferred_element_type=jnp.float32)


def _pool_body(acc_ref, dinv_ref, b_ref, batch_ref, sums_ref, cnt_ref):
    i = pl.program_id(1)
    h = jnp.maximum(dinv_ref[0] * acc_ref[0] + b_ref[...], 0.0)
    onehot = (batch_ref[0] == lax.broadcasted_iota(jnp.int32, (1, G), 1)
              ).astype(jnp.float32)
    dn = (((0,), (0,)), ((), ()))
    sums_blk = lax.dot_general(onehot, h, dn, preferred_element_type=jnp.float32)
    cnt_blk = lax.dot_general(onehot, jnp.ones((BR, H), jnp.float32), dn,
                              preferred_element_type=jnp.float32)

    @pl.when(i == 0)
    def _():
        sums_ref[0] = sums_blk
        cnt_ref[0] = cnt_blk

    @pl.when(i > 0)
    def _():
        sums_ref[0] += sums_blk
        cnt_ref[0] += cnt_blk


def _head_body(sums_ref, cnt_ref, wout_ref, bout_ref, wm1_ref, bm1_ref,
               wm2_ref, bm2_ref, out_ref):
    ga = sums_ref[0] / jnp.maximum(cnt_ref[0], 1.0)
    gb = sums_ref[1] / jnp.maximum(cnt_ref[1], 1.0)
    za = jnp.maximum(jnp.dot(ga, wout_ref[...], preferred_element_type=jnp.float32)
                     + bout_ref[...], 0.0)
    zb = jnp.maximum(jnp.dot(gb, wout_ref[...], preferred_element_type=jnp.float32)
                     + bout_ref[...], 0.0)
    hid = (jnp.dot(za, wm1_ref[0:H], preferred_element_type=jnp.float32)
           + jnp.dot(zb, wm1_ref[H:2 * H], preferred_element_type=jnp.float32)
           + jnp.dot(jnp.abs(za - zb), wm1_ref[2 * H:3 * H],
                     preferred_element_type=jnp.float32)
           + jnp.dot(za * zb, wm1_ref[3 * H:4 * H],
                     preferred_element_type=jnp.float32)
           + bm1_ref[...])
    hid = jnp.maximum(hid, 0.0)
    out_ref[...] = (jnp.dot(hid, wm2_ref[...], preferred_element_type=jnp.float32)
                    + bm2_ref[...])


def _full2(shape):
    return pl.BlockSpec(shape, lambda g, i: (0, 0))


_prep_call = pl.pallas_call(
    _prep_body,
    grid=(2, NB),
    in_specs=[
        pl.BlockSpec((1, BR, 1), lambda g, i: (g, i, 0)),
        pl.BlockSpec((1, BR, 1), lambda g, i: (g, i, 0)),
        _full2((1, H)),
        _full2((1, H)),
        _full2((H, H)),
    ],
    out_specs=[
        pl.BlockSpec((1, BR, H), lambda g, i: (g, i, 0)),
        pl.BlockSpec((1, BR, 1), lambda g, i: (g, i, 0)),
    ],
    out_shape=[
        jax.ShapeDtypeStruct((2, NPA, H), jnp.float32),
        jax.ShapeDtypeStruct((2, N, 1), jnp.float32),
    ],
    compiler_params=pltpu.CompilerParams(
        dimension_semantics=("parallel", "parallel")),
)

_mid_call = pl.pallas_call(
    _mid_body,
    grid=(2, NB),
    in_specs=[
        pl.BlockSpec((1, BR, H), lambda g, i: (g, i, 0)),
        pl.BlockSpec((1, BR, 1), lambda g, i: (g, i, 0)),
        _full2((1, H)),
        _full2((H, H)),
    ],
    out_specs=pl.BlockSpec((1, BR, H), lambda g, i: (g, i, 0)),
    out_shape=jax.ShapeDtypeStruct((2, NPA, H), jnp.float32),
    compiler_params=pltpu.CompilerParams(
        dimension_semantics=("parallel", "parallel")),
)

_pool_call = pl.pallas_call(
    _pool_body,
    grid=(2, NB),
    in_specs=[
        pl.BlockSpec((1, BR, H), lambda g, i: (g, i, 0)),
        pl.BlockSpec((1, BR, 1), lambda g, i: (g, i, 0)),
        _full2((1, H)),
        pl.BlockSpec((1, BR, 1), lambda g, i: (g, i, 0)),
    ],
    out_specs=[
        pl.BlockSpec((1, G, H), lambda g, i: (g, 0, 0)),
        pl.BlockSpec((1, G, H), lambda g, i: (g, 0, 0)),
    ],
    out_shape=[
        jax.ShapeDtypeStruct((2, G, H), jnp.float32),
        jax.ShapeDtypeStruct((2, G, H), jnp.float32),
    ],
    compiler_params=pltpu.CompilerParams(
        dimension_semantics=("parallel", "arbitrary")),
)

_head_call = pl.pallas_call(
    _head_body,
    out_shape=jax.ShapeDtypeStruct((G, 1), jnp.float32),
)


def _prep_edges(edge_index, core, e_pad, nch):
    e = edge_index.shape[1]
    pad = e_pad - e
    src = jnp.concatenate(
        [edge_index[0], jnp.zeros((pad,), jnp.int32)]) + core * NPA
    dst = jnp.concatenate([edge_index[1], jnp.full((pad,), N, jnp.int32)])
    return src.reshape(NS, nch, CHUNK), dst.reshape(NS, nch, CHUNK)


def kernel(x_a, edge_index_a, batch_a, x_b, edge_index_b, batch_b,
           W_in, b_in, W1, b1, W2, b2, W3, b3, W_out, b_out,
           W_m1, b_m1, W_m2, b_m2):
    e = edge_index_a.shape[1]
    nch = -(-e // (NS * CHUNK))
    nch = -(-nch // IDXBLK) * IDXBLK
    e_pad = NS * nch * CHUNK

    src_a, dst_a = _prep_edges(edge_index_a, 0, e_pad, nch)
    src_b, dst_b = _prep_edges(edge_index_b, 1, e_pad, nch)
    src_all = jnp.stack([src_a, src_b])
    dst_all = jnp.stack([dst_a, dst_b])

    deg = _make_deg(nch)(dst_all)
    deg_n = deg.reshape(2, NP, 1)[:, :N, :]
    x_s = jnp.stack([x_a, x_b]).reshape(2, N, 1)

    agg = _make_agg(nch)
    ts1, dinv = _prep_call(x_s, deg_n, W_in.reshape(1, H),
                           b_in.reshape(1, H), W1)
    acc1 = agg(ts1.reshape(2 * NPA, H), src_all, dst_all)
    ts2 = _mid_call(acc1.reshape(2, NPA, H), dinv, b1.reshape(1, H), W2)
    acc2 = agg(ts2.reshape(2 * NPA, H), src_all, dst_all)
    ts3 = _mid_call(acc2.reshape(2, NPA, H), dinv, b2.reshape(1, H), W3)
    acc3 = agg(ts3.reshape(2 * NPA, H), src_all, dst_all)

    batch_s = jnp.stack([batch_a, batch_b]).reshape(2, N, 1)
    sums, cnt = _pool_call(acc3.reshape(2, NPA, H), dinv,
                           b3.reshape(1, H), batch_s)
    out = _head_call(sums, cnt, W_out, b_out.reshape(1, H),
                     W_m1, b_m1.reshape(1, H), W_m2, b_m2.reshape(1, 1))
    return out.reshape(G)
